# Initial kernel scaffold; baseline (speedup 1.0000x reference)
import jax
import jax.numpy as jnp
from jax.experimental import pallas as pl


def kernel(x, pos, lattice, edge_index, enc_W, enc_b, msg_W1, msg_b1, msg_W2, msg_b2, coord_W1, coord_b1, coord_W2, coord_b2, node_W1, node_b1, node_W2, node_b2, ro_W1, ro_b1, ro_W2, ro_b2):
    raise NotImplementedError("write your pallas kernel here")


# speedup vs baseline: 4.0158x; 4.0158x over previous
import functools

import jax
import jax.numpy as jnp
from jax import lax
from jax.experimental import pallas as pl
from jax.experimental.pallas import tpu as pltpu
from jax.experimental.pallas import tpu_sc as plsc

NC = 2
NS = 16
NW = NC * NS
LANES = 16

CHI = 80
SUB = 5
ECH = CHI * SUB
BROW = 80

_F32 = jnp.float32
_I32 = jnp.int32


def _sc_mesh():
    return plsc.VectorSubcoreMesh(core_axis_name="c", subcore_axis_name="s")


def _fullf(v):
    return jnp.full((LANES,), v, _F32)


def _fulli(v):
    return jnp.full((LANES,), v, _I32)


def _sc_gather_edges(n, e, h):
    ept = e // NW
    nit = ept // ECH

    @functools.partial(
        pl.kernel,
        out_type=jax.ShapeDtypeStruct((e, h), _F32),
        mesh=_sc_mesh(),
        compiler_params=pltpu.CompilerParams(needs_layout_passes=False),
        scratch_types=[
            pltpu.VMEM((SUB, CHI), _I32),
            pltpu.VMEM((SUB, CHI), _I32),
            pltpu.VMEM((ECH, h), _F32),
            pltpu.VMEM((2 * BROW, h), _F32),
            pltpu.SemaphoreType.DMA,
            pltpu.SemaphoreType.DMA,
            pltpu.SemaphoreType.DMA,
        ],
    )
    def k(a_t, b_t, dst3, src3, g_o,
          idxd, idxs, buf_a, buf_b, sem_a, sem_b0, sem_b1):
        cid = lax.axis_index("c")
        sid = lax.axis_index("s")
        tbase = (cid * NS + sid) * ept
        sem_b = (sem_b0, sem_b1)

        def it(j, carry):
            ebase = pl.multiple_of(tbase + j * ECH, 8)
            cb = tbase // ECH + j
            pltpu.sync_copy(dst3.at[cb], idxd)
            pltpu.sync_copy(src3.at[cb], idxs)
            hs_a = []
            for s in range(SUB):
                r = pl.ds(s * CHI, CHI)
                hs_a.append(pltpu.async_copy(a_t.at[idxd.at[s]], buf_a.at[r],
                                             sem_a))
            hs_b = [pltpu.async_copy(b_t.at[idxs.at[0]], buf_b.at[pl.ds(0, BROW)],
                                     sem_b[0])]
            for hh in hs_a:
                hh.wait()
            for s in range(SUB):
                if s + 1 < SUB:
                    slot = (s + 1) % 2
                    hs_b.append(pltpu.async_copy(
                        b_t.at[idxs.at[s + 1]],
                        buf_b.at[pl.ds(slot * BROW, BROW)], sem_b[slot]))
                hs_b[s].wait()
                arow = s * CHI
                brow = (s % 2) * BROW

                def addr(r, carry2):
                    for cc in range(h // LANES):
                        sl = pl.ds(cc * LANES, LANES)
                        buf_a[arow + r, sl] = buf_a[arow + r, sl] + \
                            buf_b[brow + r, sl]
                    return carry2

                lax.fori_loop(0, CHI, addr, None, unroll=4)
            pltpu.sync_copy(buf_a, g_o.at[pl.ds(ebase, ECH)])
            return carry

        lax.fori_loop(0, nit, it, None)

    return k


def _sc_geometry(n, e):
    ept = e // NW
    nit = ept // ECH

    @functools.partial(
        pl.kernel,
        out_type=[
            jax.ShapeDtypeStruct((e, LANES), _F32),
            jax.ShapeDtypeStruct((e,), _F32),
            jax.ShapeDtypeStruct((e,), _F32),
            jax.ShapeDtypeStruct((e,), _F32),
        ],
        mesh=_sc_mesh(),
        compiler_params=pltpu.CompilerParams(needs_layout_passes=False),
        scratch_types=[
            pltpu.VMEM((SUB, CHI), _I32),
            pltpu.VMEM((SUB, CHI), _I32),
            pltpu.VMEM((ECH, LANES), _F32),
            pltpu.VMEM((ECH,), _F32),
            pltpu.VMEM((ECH,), _F32),
            pltpu.VMEM((ECH,), _F32),
            pltpu.VMEM((n,), _F32),
            pltpu.VMEM((n,), _F32),
            pltpu.VMEM((n,), _F32),
        ],
    )
    def k(px, py, pz, dst3, src3, d_o, qx_o, qy_o, qz_o,
          idxd, idxs, buf_d, buf_qx, buf_qy, buf_qz, tpx, tpy, tpz):
        cid = lax.axis_index("c")
        sid = lax.axis_index("s")
        tbase = (cid * NS + sid) * ept

        pltpu.sync_copy(px, tpx)
        pltpu.sync_copy(py, tpy)
        pltpu.sync_copy(pz, tpz)

        def zr(r, carry):
            buf_d[r, :] = _fullf(0.0)
            return carry

        lax.fori_loop(0, ECH, zr, None)

        iota16 = lax.iota(_I32, LANES)
        colz = _fulli(0)
        magic = _fulli(0x5F3759DF)
        c15 = _fullf(1.5)
        chalf = _fullf(0.5)
        one = _fullf(1.0)
        eps = _fullf(1e-8)

        def it(j, carry):
            ebase = pl.multiple_of(tbase + j * ECH, 8)
            cb = tbase // ECH + j
            pltpu.sync_copy(dst3.at[cb], idxd)
            pltpu.sync_copy(src3.at[cb], idxs)
            for g in range(ECH // LANES):
                s, o = g // SUB, (g % SUB) * LANES
                ivd = idxd[s, pl.ds(o, LANES)]
                ivs = idxs[s, pl.ds(o, LANES)]
                dx = plsc.load_gather(tpx, [ivd]) - plsc.load_gather(tpx, [ivs])
                dy = plsc.load_gather(tpy, [ivd]) - plsc.load_gather(tpy, [ivs])
                dz = plsc.load_gather(tpz, [ivd]) - plsc.load_gather(tpz, [ivs])
                d2 = dx * dx + dy * dy + dz * dz
                y = plsc.bitcast(magic - lax.shift_right_logical(
                    plsc.bitcast(d2, _I32), _fulli(1)), _F32)
                for _ in range(3):
                    y = y * (c15 - chalf * ((d2 * y) * y))
                dist = d2 * y
                rinv = one / (dist + eps)
                buf_qx[pl.ds(g * LANES, LANES)] = -dx * rinv
                buf_qy[pl.ds(g * LANES, LANES)] = -dy * rinv
                buf_qz[pl.ds(g * LANES, LANES)] = -dz * rinv
                plsc.store_scatter(buf_d, [g * LANES + iota16, colz], dist)
            pltpu.sync_copy(buf_d, d_o.at[pl.ds(ebase, ECH)])
            pltpu.sync_copy(buf_qx, qx_o.at[pl.ds(ebase, ECH)])
            pltpu.sync_copy(buf_qy, qy_o.at[pl.ds(ebase, ECH)])
            pltpu.sync_copy(buf_qz, qz_o.at[pl.ds(ebase, ECH)])
            return carry

        lax.fori_loop(0, nit, it, None)

    return k


def _sc_scatter_agg(n, e, h):
    ept = e // NW
    ech = CHI
    nit = ept // ech
    ndump = 10
    rpt = n // ndump
    nz = -(-n // (NS * ech))

    @functools.partial(
        pl.kernel,
        out_type=jax.ShapeDtypeStruct((NC, n, h), _F32),
        mesh=_sc_mesh(),
        compiler_params=pltpu.CompilerParams(needs_layout_passes=False),
        scratch_types=[
            pltpu.VMEM((1, CHI), _I32),
            pltpu.VMEM((ech, h), _F32),
            pltpu.VMEM_SHARED((n, h), _F32),
            pltpu.SemaphoreType.DMA,
        ],
    )
    def k(t2, dst3, aggp, idxd, buf, agg_sp, sem):
        cid = lax.axis_index("c")
        sid = lax.axis_index("s")
        tbase = (cid * NS + sid) * ept

        def zr(r, carry):
            for cc in range(h // LANES):
                buf[r, pl.ds(cc * LANES, LANES)] = _fullf(0.0)
            return carry

        lax.fori_loop(0, ech, zr, None)

        def zc(c, carry):
            row = pl.multiple_of((sid + c * NS) * ech, 8)

            @pl.when(row < n)
            def _():
                pltpu.sync_copy(buf, agg_sp.at[pl.ds(row, ech)])

            return carry

        lax.fori_loop(0, nz, zc, None)
        plsc.subcore_barrier()

        def it(j, carry):
            ebase = pl.multiple_of(tbase + j * ech, 8)
            cb = tbase // ech + j
            cp = pltpu.async_copy(t2.at[pl.ds(ebase, ech)], buf, sem)
            pltpu.sync_copy(dst3.at[cb], idxd)
            cp.wait()
            pltpu.sync_copy(buf, agg_sp.at[idxd.at[0]], add=True)
            return carry

        lax.fori_loop(0, nit, it, None)
        plsc.subcore_barrier()
        row0 = sid * rpt

        @pl.when(sid < ndump)
        def _():
            pltpu.sync_copy(agg_sp.at[pl.ds(row0, rpt)],
                            aggp.at[cid, pl.ds(row0, rpt)])

    return k


def _sc_pos_delta(n, e):
    ept = e // NW
    nit = ept // ECH

    @functools.partial(
        pl.kernel,
        out_type=[jax.ShapeDtypeStruct((NW * n,), _F32)] * 3,
        mesh=_sc_mesh(),
        compiler_params=pltpu.CompilerParams(needs_layout_passes=False),
        scratch_types=[
            pltpu.VMEM((SUB, CHI), _I32),
            pltpu.VMEM((ECH,), _F32),
            pltpu.VMEM((ECH,), _F32),
            pltpu.VMEM((ECH,), _F32),
            pltpu.VMEM((n,), _F32),
            pltpu.VMEM((n,), _F32),
            pltpu.VMEM((n,), _F32),
            pltpu.VMEM((n,), _F32),
            pltpu.SemaphoreType.DMA,
        ],
    )
    def k(qx, qy, qz, cm, src3, ppx, ppy, ppz,
          idxs, bqx, bqy, bqz, tcm, accx, accy, accz, sem):
        cid = lax.axis_index("c")
        sid = lax.axis_index("s")
        wid = cid * NS + sid
        tbase = wid * ept

        def zr(r, carry):
            sl = pl.ds(r * LANES, LANES)
            accx[sl] = _fullf(0.0)
            accy[sl] = _fullf(0.0)
            accz[sl] = _fullf(0.0)
            return carry

        lax.fori_loop(0, n // LANES, zr, None)
        pltpu.sync_copy(cm, tcm)

        def it(j, carry):
            ebase = pl.multiple_of(tbase + j * ECH, 8)
            cb = tbase // ECH + j
            hs = [pltpu.async_copy(qx.at[pl.ds(ebase, ECH)], bqx, sem),
                  pltpu.async_copy(qy.at[pl.ds(ebase, ECH)], bqy, sem),
                  pltpu.async_copy(qz.at[pl.ds(ebase, ECH)], bqz, sem)]
            pltpu.sync_copy(src3.at[cb], idxs)
            for hh in hs:
                hh.wait()
            for g in range(ECH // LANES):
                s, o = g // SUB, (g % SUB) * LANES
                iv = idxs[s, pl.ds(o, LANES)]
                cmv = plsc.load_gather(tcm, [iv])
                sl = pl.ds(g * LANES, LANES)
                plsc.addupdate_scatter(accx, [iv], bqx[sl] * cmv)
                plsc.addupdate_scatter(accy, [iv], bqy[sl] * cmv)
                plsc.addupdate_scatter(accz, [iv], bqz[sl] * cmv)
            return carry

        lax.fori_loop(0, nit, it, None)
        base = wid * n
        pltpu.sync_copy(accx, ppx.at[pl.ds(base, n)])
        pltpu.sync_copy(accy, ppy.at[pl.ds(base, n)])
        pltpu.sync_copy(accz, ppz.at[pl.ds(base, n)])

    return k


def _silu(x):
    return x / (1.0 + jnp.exp(-x))


def _dot(a, b):
    return jnp.dot(a, b, preferred_element_type=_F32)


def _tc_encode(n, d, h, mblk):
    def body(x_r, ew_r, eb_r, w1a_r, w1b_r, b1_r, h_r, a_r, b_r):
        hv = _dot(x_r[...], ew_r[...]) + eb_r[...]
        h_r[...] = hv
        a_r[...] = _dot(hv, w1a_r[...]) + b1_r[...]
        b_r[...] = _dot(hv, w1b_r[...])

    grid = (n // mblk,)
    bs_m = pl.BlockSpec((mblk, d), lambda i: (i, 0))
    bs_w = pl.BlockSpec((d, h), lambda i: (0, 0))
    bs_b = pl.BlockSpec((1, h), lambda i: (0, 0))
    return pl.pallas_call(
        body,
        grid=grid,
        in_specs=[bs_m, bs_w, bs_b, bs_w, bs_w, bs_b],
        out_specs=[bs_m, bs_m, bs_m],
        out_shape=[jax.ShapeDtypeStruct((n, h), _F32)] * 3,
    )


def _tc_edge(e, h, beblk):
    def body(g_r, d_r, w2_r, b2_r, w1c_r, t2_r):
        dist = d_r[...][:, :1]
        t1 = _silu(g_r[...] + dist * w1c_r[...])
        t2_r[...] = _silu(_dot(t1, w2_r[...]) + b2_r[...])

    grid = (e // beblk,)
    bs_e = pl.BlockSpec((beblk, h), lambda i: (i, 0))
    bs_p = pl.BlockSpec((beblk, LANES), lambda i: (i, 0))
    bs_w = pl.BlockSpec((h, h), lambda i: (0, 0))
    bs_b = pl.BlockSpec((1, h), lambda i: (0, 0))
    return pl.pallas_call(
        body,
        grid=grid,
        in_specs=[bs_e, bs_p, bs_w, bs_b, bs_b],
        out_specs=bs_e,
        out_shape=jax.ShapeDtypeStruct((e, h), _F32),
    )


def _tc_node(n, h, mblk, last):
    def body(h_r, ag_r, nw1h_r, nw1a_r, nb1_r, nw2_r, nb2_r, *rest):
        agg = ag_r[0] + ag_r[1]
        u = _silu(_dot(h_r[...], nw1h_r[...]) + _dot(agg, nw1a_r[...]) + nb1_r[...])
        hn = _dot(u, nw2_r[...]) + nb2_r[...]
        if last:
            hn_r, = rest
            hn_r[...] = hn
            return
        (cw1_r, cb1_r, cw2_r, cb2_r, w1an_r, w1bn_r, b1n_r,
         hn_r, cm_r, an_r, bn_r) = rest
        hn_r[...] = hn
        c1 = _silu(_dot(agg, cw1_r[...]) + cb1_r[...])
        cm_r[...] = _dot(c1, cw2_r[...]) + cb2_r[...]
        an_r[...] = _dot(hn, w1an_r[...]) + b1n_r[...]
        bn_r[...] = _dot(hn, w1bn_r[...])

    grid = (n // mblk,)
    bs_m = pl.BlockSpec((mblk, h), lambda i: (i, 0))
    bs_ag = pl.BlockSpec((NC, mblk, h), lambda i: (0, i, 0))
    bs_w = pl.BlockSpec((h, h), lambda i: (0, 0))
    bs_b = pl.BlockSpec((1, h), lambda i: (0, 0))
    bs_wc = pl.BlockSpec((h, 1), lambda i: (0, 0))
    bs_s = pl.BlockSpec((1, 1), lambda i: (0, 0))
    bs_c = pl.BlockSpec((mblk, 1), lambda i: (i, 0))
    if last:
        in_specs = [bs_m, bs_ag, bs_w, bs_w, bs_b, bs_w, bs_b]
        out_specs = bs_m
        out_shape = jax.ShapeDtypeStruct((n, h), _F32)
    else:
        in_specs = [bs_m, bs_ag, bs_w, bs_w, bs_b, bs_w, bs_b,
                    bs_w, bs_b, bs_wc, bs_s, bs_w, bs_w, bs_b]
        out_specs = [bs_m, bs_c, bs_m, bs_m]
        out_shape = [jax.ShapeDtypeStruct((n, h), _F32),
                     jax.ShapeDtypeStruct((n, 1), _F32),
                     jax.ShapeDtypeStruct((n, h), _F32),
                     jax.ShapeDtypeStruct((n, h), _F32)]
    return pl.pallas_call(body, grid=grid, in_specs=in_specs,
                          out_specs=out_specs, out_shape=out_shape)


def _tc_pos_update(n):
    rows = n // LANES

    def body(px_r, py_r, pz_r, ppx_r, ppy_r, ppz_r, ox_r, oy_r, oz_r):
        ox_r[...] = px_r[...] + jnp.sum(ppx_r[...], axis=0)
        oy_r[...] = py_r[...] + jnp.sum(ppy_r[...], axis=0)
        oz_r[...] = pz_r[...] + jnp.sum(ppz_r[...], axis=0)

    bs_p = pl.BlockSpec((rows, LANES), lambda: (0, 0))
    bs_pp = pl.BlockSpec((NW, rows, LANES), lambda: (0, 0, 0))
    return pl.pallas_call(
        body,
        in_specs=[bs_p, bs_p, bs_p, bs_pp, bs_pp, bs_pp],
        out_specs=[bs_p] * 3,
        out_shape=[jax.ShapeDtypeStruct((rows, LANES), _F32)] * 3,
    )


def _tc_readout(n, h, mblk):
    nb = n // mblk

    def body(h_r, w1_r, b1_r, w2_r, b2_r, o_r, acc_r):
        @pl.when(pl.program_id(0) == 0)
        def _():
            acc_r[...] = jnp.zeros_like(acc_r)

        acc_r[...] += jnp.sum(h_r[...], axis=0, keepdims=True)

        @pl.when(pl.program_id(0) == nb - 1)
        def _():
            g = acc_r[...] * (1.0 / n)
            z = jnp.maximum(_dot(g, w1_r[...]) + b1_r[...], 0.0)
            o_r[...] = _dot(z, w2_r[...]) + b2_r[...]

    return pl.pallas_call(
        body,
        grid=(nb,),
        in_specs=[pl.BlockSpec((mblk, h), lambda i: (i, 0)),
                  pl.BlockSpec((h, h), lambda i: (0, 0)),
                  pl.BlockSpec((1, h), lambda i: (0, 0)),
                  pl.BlockSpec((h, 1), lambda i: (0, 0)),
                  pl.BlockSpec((1, 1), lambda i: (0, 0))],
        out_specs=pl.BlockSpec((1, 1), lambda i: (0, 0)),
        out_shape=jax.ShapeDtypeStruct((1, 1), _F32),
        scratch_shapes=[pltpu.VMEM((1, h), _F32)],
        compiler_params=pltpu.CompilerParams(
            dimension_semantics=("arbitrary",)),
    )


def kernel(x, pos, lattice, edge_index, enc_W, enc_b, msg_W1, msg_b1, msg_W2,
           msg_b2, coord_W1, coord_b1, coord_W2, coord_b2, node_W1, node_b1,
           node_W2, node_b2, ro_W1, ro_b1, ro_W2, ro_b2):
    n, d = x.shape
    e = edge_index.shape[1]
    h = enc_W.shape[0]
    nlayers = msg_W1.shape[0]
    mblk = 400
    beblk = 512
    assert e % (NW * ECH) == 0 and n % mblk == 0 and n % 2000 == 0
    assert e % beblk == 0 and h % LANES == 0 and n % LANES == 0

    src = edge_index[0]
    dst = edge_index[1]
    src3 = src.reshape(e // ECH, SUB, CHI)
    dst3 = dst.reshape(e // ECH, SUB, CHI)
    dst3s = dst.reshape(e // CHI, 1, CHI)
    px = pos[:, 0]
    py = pos[:, 1]
    pz = pos[:, 2]
    prow = n // LANES

    row_b = lambda v: v.reshape(1, -1)

    gather_k = _sc_gather_edges(n, e, h)
    geom_k = _sc_geometry(n, e)
    scatter_k = _sc_scatter_agg(n, e, h)
    posdel_k = _sc_pos_delta(n, e)
    enc_k = _tc_encode(n, d, h, mblk)
    edge_k = _tc_edge(e, h, beblk)
    node_k = _tc_node(n, h, mblk, last=False)
    node_last_k = _tc_node(n, h, mblk, last=True)
    posupd_k = _tc_pos_update(n)
    ro_k = _tc_readout(n, h, mblk)

    hcur, a_t, b_t = enc_k(x, enc_W.T, row_b(enc_b),
                           msg_W1[0][:, :h].T, msg_W1[0][:, h:2 * h].T,
                           row_b(msg_b1[0]))
    for l in range(nlayers):
        g_e = gather_k(a_t, b_t, dst3, src3)
        drow, qx, qy, qz = geom_k(px, py, pz, dst3, src3)
        t2 = edge_k(g_e, drow, msg_W2[l].T, row_b(msg_b2[l]),
                    msg_W1[l][:, 2 * h].reshape(1, h))
        aggp = scatter_k(t2, dst3s)
        if l < nlayers - 1:
            hcur, cm, a_t, b_t = node_k(
                hcur, aggp,
                node_W1[l][:, :h].T, node_W1[l][:, h:].T, row_b(node_b1[l]),
                node_W2[l].T, row_b(node_b2[l]),
                coord_W1[l].T, row_b(coord_b1[l]),
                coord_W2[l].T, coord_b2[l].reshape(1, 1),
                msg_W1[l + 1][:, :h].T, msg_W1[l + 1][:, h:2 * h].T,
                row_b(msg_b1[l + 1]))
            ppx, ppy, ppz = posdel_k(qx, qy, qz, cm.reshape(n), src3)
            pxn, pyn, pzn = posupd_k(
                px.reshape(prow, LANES), py.reshape(prow, LANES),
                pz.reshape(prow, LANES),
                ppx.reshape(NW, prow, LANES), ppy.reshape(NW, prow, LANES),
                ppz.reshape(NW, prow, LANES))
            px, py, pz = pxn.reshape(n), pyn.reshape(n), pzn.reshape(n)
        else:
            hcur = node_last_k(hcur, aggp,
                               node_W1[l][:, :h].T, node_W1[l][:, h:].T,
                               row_b(node_b1[l]), node_W2[l].T,
                               row_b(node_b2[l]))
    out = ro_k(hcur, ro_W1.T, row_b(ro_b1), ro_W2.T, ro_b2.reshape(1, 1))
    return out.reshape(1)


# speedup vs baseline: 4.5132x; 1.1239x over previous
import functools

import jax
import jax.numpy as jnp
from jax import lax
from jax.experimental import pallas as pl
from jax.experimental.pallas import tpu as pltpu
from jax.experimental.pallas import tpu_sc as plsc

NC = 2
NS = 16
NW = NC * NS
LANES = 16

CHI = 80
SUB = 5
ECH = CHI * SUB
BROW = 80

_F32 = jnp.float32
_I32 = jnp.int32


def _sc_mesh():
    return plsc.VectorSubcoreMesh(core_axis_name="c", subcore_axis_name="s")


def _fullf(v):
    return jnp.full((LANES,), v, _F32)


def _fulli(v):
    return jnp.full((LANES,), v, _I32)


def _sc_gather_edges(n, e, h):
    ept = e // NW
    nit = ept // ECH

    @functools.partial(
        pl.kernel,
        out_type=jax.ShapeDtypeStruct((e, h), _F32),
        mesh=_sc_mesh(),
        compiler_params=pltpu.CompilerParams(needs_layout_passes=False),
        scratch_types=[
            pltpu.VMEM((ept,), _I32),
            pltpu.VMEM((ept,), _I32),
            pltpu.VMEM((ECH, h), _F32),
            pltpu.VMEM((2 * BROW, h), _F32),
            pltpu.SemaphoreType.DMA,
            pltpu.SemaphoreType.DMA,
            pltpu.SemaphoreType.DMA,
        ],
    )
    def k(a_t, b_t, dst_e, src_e, g_o,
          idxd, idxs, buf_a, buf_b, sem_a, sem_b0, sem_b1):
        cid = lax.axis_index("c")
        sid = lax.axis_index("s")
        wid = cid * NS + sid
        tbase = wid * ept
        sem_b = (sem_b0, sem_b1)
        pltpu.sync_copy(dst_e.at[pl.ds(tbase, ept)], idxd)
        pltpu.sync_copy(src_e.at[pl.ds(tbase, ept)], idxs)

        def it(j, carry):
            ebase = pl.multiple_of(tbase + j * ECH, 8)
            hs_a = []
            for s in range(SUB):
                r = pl.ds(s * CHI, CHI)
                hs_a.append(pltpu.async_copy(a_t.at[idxd.at[pl.ds(j * ECH + s * CHI, CHI)]], buf_a.at[r],
                                             sem_a))
            hs_b = [pltpu.async_copy(b_t.at[idxs.at[pl.ds(j * ECH, CHI)]],
                                     buf_b.at[pl.ds(0, BROW)], sem_b[0])]
            for hh in hs_a:
                hh.wait()
            for s in range(SUB):
                if s + 1 < SUB:
                    slot = (s + 1) % 2
                    hs_b.append(pltpu.async_copy(
                        b_t.at[idxs.at[pl.ds(j * ECH + (s + 1) * CHI, CHI)]],
                        buf_b.at[pl.ds(slot * BROW, BROW)], sem_b[slot]))
                hs_b[s].wait()
                arow = s * CHI
                brow = (s % 2) * BROW

                def addr(r, carry2):
                    for cc in range(h // LANES):
                        sl = pl.ds(cc * LANES, LANES)
                        buf_a[arow + r, sl] = buf_a[arow + r, sl] + \
                            buf_b[brow + r, sl]
                    return carry2

                lax.fori_loop(0, CHI, addr, None, unroll=4)
            pltpu.sync_copy(buf_a, g_o.at[pl.ds(ebase, ECH)])
            return carry

        lax.fori_loop(0, nit, it, None)

    return k


def _sc_geometry(n, e):
    ept = e // NW
    nit = ept // ECH

    @functools.partial(
        pl.kernel,
        out_type=[
            jax.ShapeDtypeStruct((e, LANES), _F32),
            jax.ShapeDtypeStruct((e,), _F32),
            jax.ShapeDtypeStruct((e,), _F32),
            jax.ShapeDtypeStruct((e,), _F32),
        ],
        mesh=_sc_mesh(),
        compiler_params=pltpu.CompilerParams(needs_layout_passes=False),
        scratch_types=[
            pltpu.VMEM((ept,), _I32),
            pltpu.VMEM((ept,), _I32),
            pltpu.VMEM((ECH, LANES), _F32),
            pltpu.VMEM((ECH,), _F32),
            pltpu.VMEM((ECH,), _F32),
            pltpu.VMEM((ECH,), _F32),
            pltpu.VMEM((n,), _F32),
            pltpu.VMEM((n,), _F32),
            pltpu.VMEM((n,), _F32),
        ],
    )
    def k(px, py, pz, dst_e, src_e, d_o, qx_o, qy_o, qz_o,
          idxd, idxs, buf_d, buf_qx, buf_qy, buf_qz, tpx, tpy, tpz):
        cid = lax.axis_index("c")
        sid = lax.axis_index("s")
        wid = cid * NS + sid
        tbase = wid * ept

        pltpu.sync_copy(px, tpx)
        pltpu.sync_copy(py, tpy)
        pltpu.sync_copy(pz, tpz)
        pltpu.sync_copy(dst_e.at[pl.ds(tbase, ept)], idxd)
        pltpu.sync_copy(src_e.at[pl.ds(tbase, ept)], idxs)

        def zr(r, carry):
            buf_d[r, :] = _fullf(0.0)
            return carry

        lax.fori_loop(0, ECH, zr, None)

        iota16 = lax.iota(_I32, LANES)
        colz = _fulli(0)
        magic = _fulli(0x5F3759DF)
        c15 = _fullf(1.5)
        chalf = _fullf(0.5)
        one = _fullf(1.0)
        eps = _fullf(1e-8)

        def it(j, carry):
            ebase = pl.multiple_of(tbase + j * ECH, 8)
            for g in range(ECH // LANES):
                ivd = idxd[pl.ds(j * ECH + g * LANES, LANES)]
                ivs = idxs[pl.ds(j * ECH + g * LANES, LANES)]
                dx = plsc.load_gather(tpx, [ivd]) - plsc.load_gather(tpx, [ivs])
                dy = plsc.load_gather(tpy, [ivd]) - plsc.load_gather(tpy, [ivs])
                dz = plsc.load_gather(tpz, [ivd]) - plsc.load_gather(tpz, [ivs])
                d2 = dx * dx + dy * dy + dz * dz
                y = plsc.bitcast(magic - lax.shift_right_logical(
                    plsc.bitcast(d2, _I32), _fulli(1)), _F32)
                for _ in range(3):
                    y = y * (c15 - chalf * ((d2 * y) * y))
                dist = d2 * y
                rinv = one / (dist + eps)
                buf_qx[pl.ds(g * LANES, LANES)] = -dx * rinv
                buf_qy[pl.ds(g * LANES, LANES)] = -dy * rinv
                buf_qz[pl.ds(g * LANES, LANES)] = -dz * rinv
                plsc.store_scatter(buf_d, [g * LANES + iota16, colz], dist)
            pltpu.sync_copy(buf_d, d_o.at[pl.ds(ebase, ECH)])
            pltpu.sync_copy(buf_qx, qx_o.at[pl.ds(ebase, ECH)])
            pltpu.sync_copy(buf_qy, qy_o.at[pl.ds(ebase, ECH)])
            pltpu.sync_copy(buf_qz, qz_o.at[pl.ds(ebase, ECH)])
            return carry

        lax.fori_loop(0, nit, it, None)

    return k


def _sc_scatter_agg(n, e, h):
    ept = e // NW
    ech = CHI
    nit = ept // ech
    ndump = 10
    rpt = n // ndump
    nz = -(-n // (NS * ech))

    @functools.partial(
        pl.kernel,
        out_type=jax.ShapeDtypeStruct((NC, n, h), _F32),
        mesh=_sc_mesh(),
        compiler_params=pltpu.CompilerParams(needs_layout_passes=False),
        scratch_types=[
            pltpu.VMEM((nit, CHI), _I32),
            pltpu.VMEM((2, ech, h), _F32),
            pltpu.VMEM_SHARED((n, h), _F32),
            pltpu.SemaphoreType.DMA,
            pltpu.SemaphoreType.DMA,
        ],
    )
    def k(t2, dst4, aggp, idxa, buf, agg_sp, sem0, sem1):
        cid = lax.axis_index("c")
        sid = lax.axis_index("s")
        wid = cid * NS + sid
        tbase = wid * ept
        sems = (sem0, sem1)
        pltpu.sync_copy(dst4.at[wid], idxa)

        def zr(r, carry):
            for cc in range(h // LANES):
                buf[0, r, pl.ds(cc * LANES, LANES)] = _fullf(0.0)
            return carry

        lax.fori_loop(0, ech, zr, None)

        def zc(c, carry):
            row = pl.multiple_of((sid + c * NS) * ech, 8)

            @pl.when(row < n)
            def _():
                pltpu.sync_copy(buf.at[0], agg_sp.at[pl.ds(row, ech)])

            return carry

        lax.fori_loop(0, nz, zc, None)
        plsc.subcore_barrier()

        def start(j, slot):
            ebase = pl.multiple_of(tbase + j * ech, 8)
            pltpu.async_copy(t2.at[pl.ds(ebase, ech)], buf.at[slot], sems[slot])

        def finish(j, slot):
            ebase = pl.multiple_of(tbase + j * ech, 8)
            pltpu.make_async_copy(t2.at[pl.ds(ebase, ech)], buf.at[slot],
                                  sems[slot]).wait()
            pltpu.sync_copy(buf.at[slot], agg_sp.at[idxa.at[j]], add=True)

        start(0, 0)

        def it(jj, carry):
            j0 = jj * 2

            @pl.when(j0 + 1 < nit)
            def _():
                start(j0 + 1, 1)

            finish(j0, 0)

            @pl.when(j0 + 2 < nit)
            def _():
                start(j0 + 2, 0)

            @pl.when(j0 + 1 < nit)
            def _():
                finish(j0 + 1, 1)

            return carry

        lax.fori_loop(0, (nit + 1) // 2, it, None)
        plsc.subcore_barrier()
        row0 = sid * rpt

        @pl.when(sid < ndump)
        def _():
            pltpu.sync_copy(agg_sp.at[pl.ds(row0, rpt)],
                            aggp.at[cid, pl.ds(row0, rpt)])

    return k


def _sc_pos_delta(n, e):
    ept = e // NW
    nit = ept // ECH

    @functools.partial(
        pl.kernel,
        out_type=[jax.ShapeDtypeStruct((NW * n,), _F32)] * 3,
        mesh=_sc_mesh(),
        compiler_params=pltpu.CompilerParams(needs_layout_passes=False),
        scratch_types=[
            pltpu.VMEM((ept,), _I32),
            pltpu.VMEM((ECH,), _F32),
            pltpu.VMEM((ECH,), _F32),
            pltpu.VMEM((ECH,), _F32),
            pltpu.VMEM((n,), _F32),
            pltpu.VMEM((n,), _F32),
            pltpu.VMEM((n,), _F32),
            pltpu.VMEM((n,), _F32),
            pltpu.SemaphoreType.DMA,
        ],
    )
    def k(qx, qy, qz, cm, src_e, ppx, ppy, ppz,
          idxs, bqx, bqy, bqz, tcm, accx, accy, accz, sem):
        cid = lax.axis_index("c")
        sid = lax.axis_index("s")
        wid = cid * NS + sid
        tbase = wid * ept
        pltpu.sync_copy(src_e.at[pl.ds(tbase, ept)], idxs)

        def zr(r, carry):
            sl = pl.ds(r * LANES, LANES)
            accx[sl] = _fullf(0.0)
            accy[sl] = _fullf(0.0)
            accz[sl] = _fullf(0.0)
            return carry

        lax.fori_loop(0, n // LANES, zr, None)
        pltpu.sync_copy(cm, tcm)

        def it(j, carry):
            ebase = pl.multiple_of(tbase + j * ECH, 8)
            hs = [pltpu.async_copy(qx.at[pl.ds(ebase, ECH)], bqx, sem),
                  pltpu.async_copy(qy.at[pl.ds(ebase, ECH)], bqy, sem),
                  pltpu.async_copy(qz.at[pl.ds(ebase, ECH)], bqz, sem)]
            for hh in hs:
                hh.wait()
            for g in range(ECH // LANES):
                iv = idxs[pl.ds(j * ECH + g * LANES, LANES)]
                cmv = plsc.load_gather(tcm, [iv])
                sl = pl.ds(g * LANES, LANES)
                plsc.addupdate_scatter(accx, [iv], bqx[sl] * cmv)
                plsc.addupdate_scatter(accy, [iv], bqy[sl] * cmv)
                plsc.addupdate_scatter(accz, [iv], bqz[sl] * cmv)
            return carry

        lax.fori_loop(0, nit, it, None)
        base = wid * n
        pltpu.sync_copy(accx, ppx.at[pl.ds(base, n)])
        pltpu.sync_copy(accy, ppy.at[pl.ds(base, n)])
        pltpu.sync_copy(accz, ppz.at[pl.ds(base, n)])

    return k


def _silu(x):
    return x / (1.0 + jnp.exp(-x))


def _dot(a, b):
    return jnp.dot(a, b, preferred_element_type=_F32)


def _tc_encode(n, d, h, mblk):
    def body(x_r, ew_r, eb_r, w1a_r, w1b_r, b1_r, h_r, a_r, b_r):
        hv = _dot(x_r[...], ew_r[...]) + eb_r[...]
        h_r[...] = hv
        a_r[...] = _dot(hv, w1a_r[...]) + b1_r[...]
        b_r[...] = _dot(hv, w1b_r[...])

    grid = (n // mblk,)
    bs_m = pl.BlockSpec((mblk, d), lambda i: (i, 0))
    bs_w = pl.BlockSpec((d, h), lambda i: (0, 0))
    bs_b = pl.BlockSpec((1, h), lambda i: (0, 0))
    return pl.pallas_call(
        body,
        grid=grid,
        in_specs=[bs_m, bs_w, bs_b, bs_w, bs_w, bs_b],
        out_specs=[bs_m, bs_m, bs_m],
        out_shape=[jax.ShapeDtypeStruct((n, h), _F32)] * 3,
    )


def _tc_edge(e, h, beblk):
    def body(g_r, d_r, w2_r, b2_r, w1c_r, t2_r):
        dist = d_r[...][:, :1]
        t1 = _silu(g_r[...] + dist * w1c_r[...])
        t2_r[...] = _silu(_dot(t1, w2_r[...]) + b2_r[...])

    grid = (e // beblk,)
    bs_e = pl.BlockSpec((beblk, h), lambda i: (i, 0))
    bs_p = pl.BlockSpec((beblk, LANES), lambda i: (i, 0))
    bs_w = pl.BlockSpec((h, h), lambda i: (0, 0))
    bs_b = pl.BlockSpec((1, h), lambda i: (0, 0))
    return pl.pallas_call(
        body,
        grid=grid,
        in_specs=[bs_e, bs_p, bs_w, bs_b, bs_b],
        out_specs=bs_e,
        out_shape=jax.ShapeDtypeStruct((e, h), _F32),
    )


def _tc_node(n, h, mblk, last):
    def body(h_r, ag_r, nw1h_r, nw1a_r, nb1_r, nw2_r, nb2_r, *rest):
        agg = ag_r[0] + ag_r[1]
        u = _silu(_dot(h_r[...], nw1h_r[...]) + _dot(agg, nw1a_r[...]) + nb1_r[...])
        hn = _dot(u, nw2_r[...]) + nb2_r[...]
        if last:
            hn_r, = rest
            hn_r[...] = hn
            return
        (cw1_r, cb1_r, cw2_r, cb2_r, w1an_r, w1bn_r, b1n_r,
         hn_r, cm_r, an_r, bn_r) = rest
        hn_r[...] = hn
        c1 = _silu(_dot(agg, cw1_r[...]) + cb1_r[...])
        cm_r[...] = _dot(c1, cw2_r[...]) + cb2_r[...]
        an_r[...] = _dot(hn, w1an_r[...]) + b1n_r[...]
        bn_r[...] = _dot(hn, w1bn_r[...])

    grid = (n // mblk,)
    bs_m = pl.BlockSpec((mblk, h), lambda i: (i, 0))
    bs_ag = pl.BlockSpec((NC, mblk, h), lambda i: (0, i, 0))
    bs_w = pl.BlockSpec((h, h), lambda i: (0, 0))
    bs_b = pl.BlockSpec((1, h), lambda i: (0, 0))
    bs_wc = pl.BlockSpec((h, 1), lambda i: (0, 0))
    bs_s = pl.BlockSpec((1, 1), lambda i: (0, 0))
    bs_c = pl.BlockSpec((mblk, 1), lambda i: (i, 0))
    if last:
        in_specs = [bs_m, bs_ag, bs_w, bs_w, bs_b, bs_w, bs_b]
        out_specs = bs_m
        out_shape = jax.ShapeDtypeStruct((n, h), _F32)
    else:
        in_specs = [bs_m, bs_ag, bs_w, bs_w, bs_b, bs_w, bs_b,
                    bs_w, bs_b, bs_wc, bs_s, bs_w, bs_w, bs_b]
        out_specs = [bs_m, bs_c, bs_m, bs_m]
        out_shape = [jax.ShapeDtypeStruct((n, h), _F32),
                     jax.ShapeDtypeStruct((n, 1), _F32),
                     jax.ShapeDtypeStruct((n, h), _F32),
                     jax.ShapeDtypeStruct((n, h), _F32)]
    return pl.pallas_call(body, grid=grid, in_specs=in_specs,
                          out_specs=out_specs, out_shape=out_shape)


def _tc_pos_update(n):
    rows = n // LANES

    def body(px_r, py_r, pz_r, ppx_r, ppy_r, ppz_r, ox_r, oy_r, oz_r):
        ox_r[...] = px_r[...] + jnp.sum(ppx_r[...], axis=0)
        oy_r[...] = py_r[...] + jnp.sum(ppy_r[...], axis=0)
        oz_r[...] = pz_r[...] + jnp.sum(ppz_r[...], axis=0)

    bs_p = pl.BlockSpec((rows, LANES), lambda: (0, 0))
    bs_pp = pl.BlockSpec((NW, rows, LANES), lambda: (0, 0, 0))
    return pl.pallas_call(
        body,
        in_specs=[bs_p, bs_p, bs_p, bs_pp, bs_pp, bs_pp],
        out_specs=[bs_p] * 3,
        out_shape=[jax.ShapeDtypeStruct((rows, LANES), _F32)] * 3,
    )


def _tc_readout(n, h, mblk):
    nb = n // mblk

    def body(h_r, w1_r, b1_r, w2_r, b2_r, o_r, acc_r):
        @pl.when(pl.program_id(0) == 0)
        def _():
            acc_r[...] = jnp.zeros_like(acc_r)

        acc_r[...] += jnp.sum(h_r[...], axis=0, keepdims=True)

        @pl.when(pl.program_id(0) == nb - 1)
        def _():
            g = acc_r[...] * (1.0 / n)
            z = jnp.maximum(_dot(g, w1_r[...]) + b1_r[...], 0.0)
            o_r[...] = _dot(z, w2_r[...]) + b2_r[...]

    return pl.pallas_call(
        body,
        grid=(nb,),
        in_specs=[pl.BlockSpec((mblk, h), lambda i: (i, 0)),
                  pl.BlockSpec((h, h), lambda i: (0, 0)),
                  pl.BlockSpec((1, h), lambda i: (0, 0)),
                  pl.BlockSpec((h, 1), lambda i: (0, 0)),
                  pl.BlockSpec((1, 1), lambda i: (0, 0))],
        out_specs=pl.BlockSpec((1, 1), lambda i: (0, 0)),
        out_shape=jax.ShapeDtypeStruct((1, 1), _F32),
        scratch_shapes=[pltpu.VMEM((1, h), _F32)],
        compiler_params=pltpu.CompilerParams(
            dimension_semantics=("arbitrary",)),
    )


def kernel(x, pos, lattice, edge_index, enc_W, enc_b, msg_W1, msg_b1, msg_W2,
           msg_b2, coord_W1, coord_b1, coord_W2, coord_b2, node_W1, node_b1,
           node_W2, node_b2, ro_W1, ro_b1, ro_W2, ro_b2):
    n, d = x.shape
    e = edge_index.shape[1]
    h = enc_W.shape[0]
    nlayers = msg_W1.shape[0]
    mblk = 400
    beblk = 512
    assert e % (NW * ECH) == 0 and n % mblk == 0 and n % 2000 == 0
    assert e % beblk == 0 and h % LANES == 0 and n % LANES == 0

    src = edge_index[0]
    dst = edge_index[1]
    dst4s = dst.reshape(NW, e // (NW * CHI), CHI)
    px = pos[:, 0]
    py = pos[:, 1]
    pz = pos[:, 2]
    prow = n // LANES

    row_b = lambda v: v.reshape(1, -1)

    gather_k = _sc_gather_edges(n, e, h)
    geom_k = _sc_geometry(n, e)
    scatter_k = _sc_scatter_agg(n, e, h)
    posdel_k = _sc_pos_delta(n, e)
    enc_k = _tc_encode(n, d, h, mblk)
    edge_k = _tc_edge(e, h, beblk)
    node_k = _tc_node(n, h, mblk, last=False)
    node_last_k = _tc_node(n, h, mblk, last=True)
    posupd_k = _tc_pos_update(n)
    ro_k = _tc_readout(n, h, mblk)

    hcur, a_t, b_t = enc_k(x, enc_W.T, row_b(enc_b),
                           msg_W1[0][:, :h].T, msg_W1[0][:, h:2 * h].T,
                           row_b(msg_b1[0]))
    for l in range(nlayers):
        g_e = gather_k(a_t, b_t, dst, src)
        drow, qx, qy, qz = geom_k(px, py, pz, dst, src)
        t2 = edge_k(g_e, drow, msg_W2[l].T, row_b(msg_b2[l]),
                    msg_W1[l][:, 2 * h].reshape(1, h))
        aggp = scatter_k(t2, dst4s)
        if l < nlayers - 1:
            hcur, cm, a_t, b_t = node_k(
                hcur, aggp,
                node_W1[l][:, :h].T, node_W1[l][:, h:].T, row_b(node_b1[l]),
                node_W2[l].T, row_b(node_b2[l]),
                coord_W1[l].T, row_b(coord_b1[l]),
                coord_W2[l].T, coord_b2[l].reshape(1, 1),
                msg_W1[l + 1][:, :h].T, msg_W1[l + 1][:, h:2 * h].T,
                row_b(msg_b1[l + 1]))
            ppx, ppy, ppz = posdel_k(qx, qy, qz, cm.reshape(n), src)
            pxn, pyn, pzn = posupd_k(
                px.reshape(prow, LANES), py.reshape(prow, LANES),
                pz.reshape(prow, LANES),
                ppx.reshape(NW, prow, LANES), ppy.reshape(NW, prow, LANES),
                ppz.reshape(NW, prow, LANES))
            px, py, pz = pxn.reshape(n), pyn.reshape(n), pzn.reshape(n)
        else:
            hcur = node_last_k(hcur, aggp,
                               node_W1[l][:, :h].T, node_W1[l][:, h:].T,
                               row_b(node_b1[l]), node_W2[l].T,
                               row_b(node_b2[l]))
    out = ro_k(hcur, ro_W1.T, row_b(ro_b1), ro_W2.T, ro_b2.reshape(1, 1))
    return out.reshape(1)


# speedup vs baseline: 4.5894x; 1.0169x over previous
import functools

import jax
import jax.numpy as jnp
from jax import lax
from jax.experimental import pallas as pl
from jax.experimental.pallas import tpu as pltpu
from jax.experimental.pallas import tpu_sc as plsc

NC = 2
NS = 16
NW = NC * NS
LANES = 16

CHI = 80
SUB = 5
ECH = CHI * SUB
BROW = 80

_F32 = jnp.float32
_I32 = jnp.int32


def _sc_mesh():
    return plsc.VectorSubcoreMesh(core_axis_name="c", subcore_axis_name="s")


def _fullf(v):
    return jnp.full((LANES,), v, _F32)


def _fulli(v):
    return jnp.full((LANES,), v, _I32)


def _sc_gather_edges(n, e, h):
    ept = e // NW
    nit = ept // ECH

    @functools.partial(
        pl.kernel,
        out_type=[
            jax.ShapeDtypeStruct((e, h), _F32),
            jax.ShapeDtypeStruct((e, h), _F32),
        ],
        mesh=_sc_mesh(),
        compiler_params=pltpu.CompilerParams(needs_layout_passes=False),
        scratch_types=[
            pltpu.VMEM((SUB, CHI), _I32),
            pltpu.VMEM((SUB, CHI), _I32),
            pltpu.VMEM((ECH, h), _F32),
            pltpu.VMEM((ECH, h), _F32),
            pltpu.SemaphoreType.DMA,
            pltpu.SemaphoreType.DMA,
            pltpu.SemaphoreType.DMA,
        ],
    )
    def k(a_t, b_t, dst3, src3, gd_o, gs_o,
          idxd, idxs, buf_a, buf_b, sem_i, sem_g, sem_s):
        cid = lax.axis_index("c")
        sid = lax.axis_index("s")
        wid = cid * NS + sid
        tbase = wid * ept

        def it(j, carry):
            ebase = pl.multiple_of(tbase + j * ECH, 8)
            cb = tbase // ECH + j
            c1 = pltpu.async_copy(dst3.at[cb], idxd, sem_i)
            c2 = pltpu.async_copy(src3.at[cb], idxs, sem_i)
            c1.wait()
            c2.wait()
            hs = []
            for s in range(SUB):
                r = pl.ds(s * CHI, CHI)
                hs.append(pltpu.async_copy(a_t.at[idxd.at[s]], buf_a.at[r],
                                           sem_g))
                hs.append(pltpu.async_copy(b_t.at[idxs.at[s]], buf_b.at[r],
                                           sem_g))
            for hh in hs:
                hh.wait()
            pltpu.sync_copy(buf_a, gd_o.at[pl.ds(ebase, ECH)])
            pltpu.sync_copy(buf_b, gs_o.at[pl.ds(ebase, ECH)])
            return carry

        lax.fori_loop(0, nit, it, None)

    return k


def _sc_geometry(n, e):
    ept = e // NW
    nit = ept // ECH

    @functools.partial(
        pl.kernel,
        out_type=[
            jax.ShapeDtypeStruct((e, LANES), _F32),
            jax.ShapeDtypeStruct((e,), _F32),
            jax.ShapeDtypeStruct((e,), _F32),
            jax.ShapeDtypeStruct((e,), _F32),
        ],
        mesh=_sc_mesh(),
        compiler_params=pltpu.CompilerParams(needs_layout_passes=False),
        scratch_types=[
            pltpu.VMEM((ept,), _I32),
            pltpu.VMEM((ept,), _I32),
            pltpu.VMEM((ECH, LANES), _F32),
            pltpu.VMEM((ECH,), _F32),
            pltpu.VMEM((ECH,), _F32),
            pltpu.VMEM((ECH,), _F32),
            pltpu.VMEM((n,), _F32),
            pltpu.VMEM((n,), _F32),
            pltpu.VMEM((n,), _F32),
        ],
    )
    def k(px, py, pz, dst_e, src_e, d_o, qx_o, qy_o, qz_o,
          idxd, idxs, buf_d, buf_qx, buf_qy, buf_qz, tpx, tpy, tpz):
        cid = lax.axis_index("c")
        sid = lax.axis_index("s")
        wid = cid * NS + sid
        tbase = wid * ept

        pltpu.sync_copy(px, tpx)
        pltpu.sync_copy(py, tpy)
        pltpu.sync_copy(pz, tpz)
        pltpu.sync_copy(dst_e.at[pl.ds(tbase, ept)], idxd)
        pltpu.sync_copy(src_e.at[pl.ds(tbase, ept)], idxs)

        def zr(r, carry):
            buf_d[r, :] = _fullf(0.0)
            return carry

        lax.fori_loop(0, ECH, zr, None)

        iota16 = lax.iota(_I32, LANES)
        colz = _fulli(0)
        magic = _fulli(0x5F3759DF)
        c15 = _fullf(1.5)
        chalf = _fullf(0.5)
        one = _fullf(1.0)
        eps = _fullf(1e-8)

        def it(j, carry):
            ebase = pl.multiple_of(tbase + j * ECH, 8)
            for g in range(ECH // LANES):
                ivd = idxd[pl.ds(j * ECH + g * LANES, LANES)]
                ivs = idxs[pl.ds(j * ECH + g * LANES, LANES)]
                dx = plsc.load_gather(tpx, [ivd]) - plsc.load_gather(tpx, [ivs])
                dy = plsc.load_gather(tpy, [ivd]) - plsc.load_gather(tpy, [ivs])
                dz = plsc.load_gather(tpz, [ivd]) - plsc.load_gather(tpz, [ivs])
                d2 = dx * dx + dy * dy + dz * dz
                y = plsc.bitcast(magic - lax.shift_right_logical(
                    plsc.bitcast(d2, _I32), _fulli(1)), _F32)
                for _ in range(3):
                    y = y * (c15 - chalf * ((d2 * y) * y))
                dist = d2 * y
                rinv = one / (dist + eps)
                buf_qx[pl.ds(g * LANES, LANES)] = -dx * rinv
                buf_qy[pl.ds(g * LANES, LANES)] = -dy * rinv
                buf_qz[pl.ds(g * LANES, LANES)] = -dz * rinv
                plsc.store_scatter(buf_d, [g * LANES + iota16, colz], d2)
            pltpu.sync_copy(buf_d, d_o.at[pl.ds(ebase, ECH)])
            pltpu.sync_copy(buf_qx, qx_o.at[pl.ds(ebase, ECH)])
            pltpu.sync_copy(buf_qy, qy_o.at[pl.ds(ebase, ECH)])
            pltpu.sync_copy(buf_qz, qz_o.at[pl.ds(ebase, ECH)])
            return carry

        lax.fori_loop(0, nit, it, None)

    return k


def _sc_scatter_agg(n, e, h):
    ept = e // NW
    ech = CHI
    nit = ept // ech
    ndump = 10
    rpt = n // ndump
    nz = -(-n // (NS * ech))

    @functools.partial(
        pl.kernel,
        out_type=jax.ShapeDtypeStruct((NC, n, h), _F32),
        mesh=_sc_mesh(),
        compiler_params=pltpu.CompilerParams(needs_layout_passes=False),
        scratch_types=[
            pltpu.VMEM((nit, CHI), _I32),
            pltpu.VMEM((2, ech, h), _F32),
            pltpu.VMEM_SHARED((n, h), _F32),
            pltpu.SemaphoreType.DMA,
            pltpu.SemaphoreType.DMA,
        ],
    )
    def k(t2, dst4, aggp, idxa, buf, agg_sp, sem0, sem1):
        cid = lax.axis_index("c")
        sid = lax.axis_index("s")
        wid = cid * NS + sid
        tbase = wid * ept
        sems = (sem0, sem1)
        pltpu.sync_copy(dst4.at[wid], idxa)

        def zr(r, carry):
            for cc in range(h // LANES):
                buf[0, r, pl.ds(cc * LANES, LANES)] = _fullf(0.0)
            return carry

        lax.fori_loop(0, ech, zr, None)

        def zc(c, carry):
            row = pl.multiple_of((sid + c * NS) * ech, 8)

            @pl.when(row < n)
            def _():
                pltpu.sync_copy(buf.at[0], agg_sp.at[pl.ds(row, ech)])

            return carry

        lax.fori_loop(0, nz, zc, None)
        plsc.subcore_barrier()

        def start(j, slot):
            ebase = pl.multiple_of(tbase + j * ech, 8)
            pltpu.async_copy(t2.at[pl.ds(ebase, ech)], buf.at[slot], sems[slot])

        def finish(j, slot):
            ebase = pl.multiple_of(tbase + j * ech, 8)
            pltpu.make_async_copy(t2.at[pl.ds(ebase, ech)], buf.at[slot],
                                  sems[slot]).wait()
            pltpu.sync_copy(buf.at[slot], agg_sp.at[idxa.at[j]], add=True)

        start(0, 0)

        def it(jj, carry):
            j0 = jj * 2

            @pl.when(j0 + 1 < nit)
            def _():
                start(j0 + 1, 1)

            finish(j0, 0)

            @pl.when(j0 + 2 < nit)
            def _():
                start(j0 + 2, 0)

            @pl.when(j0 + 1 < nit)
            def _():
                finish(j0 + 1, 1)

            return carry

        lax.fori_loop(0, (nit + 1) // 2, it, None)
        plsc.subcore_barrier()
        row0 = sid * rpt

        @pl.when(sid < ndump)
        def _():
            pltpu.sync_copy(agg_sp.at[pl.ds(row0, rpt)],
                            aggp.at[cid, pl.ds(row0, rpt)])

    return k


def _sc_pos_delta(n, e):
    ept = e // NW
    nit = ept // ECH

    @functools.partial(
        pl.kernel,
        out_type=[jax.ShapeDtypeStruct((NW * n,), _F32)] * 3,
        mesh=_sc_mesh(),
        compiler_params=pltpu.CompilerParams(needs_layout_passes=False),
        scratch_types=[
            pltpu.VMEM((ept,), _I32),
            pltpu.VMEM((ECH,), _F32),
            pltpu.VMEM((ECH,), _F32),
            pltpu.VMEM((ECH,), _F32),
            pltpu.VMEM((n,), _F32),
            pltpu.VMEM((n,), _F32),
            pltpu.VMEM((n,), _F32),
            pltpu.VMEM((n,), _F32),
            pltpu.SemaphoreType.DMA,
        ],
    )
    def k(qx, qy, qz, cm, src_e, ppx, ppy, ppz,
          idxs, bqx, bqy, bqz, tcm, accx, accy, accz, sem):
        cid = lax.axis_index("c")
        sid = lax.axis_index("s")
        wid = cid * NS + sid
        tbase = wid * ept
        pltpu.sync_copy(src_e.at[pl.ds(tbase, ept)], idxs)

        def zr(r, carry):
            sl = pl.ds(r * LANES, LANES)
            accx[sl] = _fullf(0.0)
            accy[sl] = _fullf(0.0)
            accz[sl] = _fullf(0.0)
            return carry

        lax.fori_loop(0, n // LANES, zr, None)
        pltpu.sync_copy(cm, tcm)

        def it(j, carry):
            ebase = pl.multiple_of(tbase + j * ECH, 8)
            hs = [pltpu.async_copy(qx.at[pl.ds(ebase, ECH)], bqx, sem),
                  pltpu.async_copy(qy.at[pl.ds(ebase, ECH)], bqy, sem),
                  pltpu.async_copy(qz.at[pl.ds(ebase, ECH)], bqz, sem)]
            for hh in hs:
                hh.wait()
            for g in range(ECH // LANES):
                iv = idxs[pl.ds(j * ECH + g * LANES, LANES)]
                cmv = plsc.load_gather(tcm, [iv])
                sl = pl.ds(g * LANES, LANES)
                plsc.addupdate_scatter(accx, [iv], bqx[sl] * cmv)
                plsc.addupdate_scatter(accy, [iv], bqy[sl] * cmv)
                plsc.addupdate_scatter(accz, [iv], bqz[sl] * cmv)
            return carry

        lax.fori_loop(0, nit, it, None)
        base = wid * n
        pltpu.sync_copy(accx, ppx.at[pl.ds(base, n)])
        pltpu.sync_copy(accy, ppy.at[pl.ds(base, n)])
        pltpu.sync_copy(accz, ppz.at[pl.ds(base, n)])

    return k


def _silu(x):
    return x * jax.nn.sigmoid(x)


def _dot(a, b):
    return jnp.dot(a, b, preferred_element_type=_F32)


def _tc_encode(n, d, h, mblk):
    def body(x_r, ew_r, eb_r, h_r):
        h_r[...] = _dot(x_r[...], ew_r[...]) + eb_r[...]

    grid = (n // mblk,)
    bs_m = pl.BlockSpec((mblk, d), lambda i: (i, 0))
    bs_w = pl.BlockSpec((d, h), lambda i: (0, 0))
    bs_b = pl.BlockSpec((1, h), lambda i: (0, 0))
    return pl.pallas_call(
        body,
        grid=grid,
        in_specs=[bs_m, bs_w, bs_b],
        out_specs=bs_m,
        out_shape=jax.ShapeDtypeStruct((n, h), _F32),
    )


def _tc_edge(e, h, beblk):
    def body(gd_r, gs_r, d_r, w1p_r, b1_r, w2_r, b2_r, t2_r):
        dist = jnp.sqrt(d_r[...][:, :1])
        dpad = jnp.pad(dist, ((0, 0), (0, h - 1)))
        mm = jnp.concatenate([gd_r[...], gs_r[...], dpad], axis=1)
        t1 = _silu(_dot(mm, w1p_r[...]) + b1_r[...])
        t2_r[...] = _silu(_dot(t1, w2_r[...]) + b2_r[...])

    grid = (e // beblk,)
    bs_e = pl.BlockSpec((beblk, h), lambda i: (i, 0))
    bs_p = pl.BlockSpec((beblk, LANES), lambda i: (i, 0))
    bs_w1 = pl.BlockSpec((3 * h, h), lambda i: (0, 0))
    bs_w = pl.BlockSpec((h, h), lambda i: (0, 0))
    bs_b = pl.BlockSpec((1, h), lambda i: (0, 0))
    return pl.pallas_call(
        body,
        grid=grid,
        in_specs=[bs_e, bs_e, bs_p, bs_w1, bs_b, bs_w, bs_b],
        out_specs=bs_e,
        out_shape=jax.ShapeDtypeStruct((e, h), _F32),
    )


def _tc_node(n, h, mblk, last):
    def body(h_r, ag_r, nw1_r, nb1_r, nw2_r, nb2_r, *rest):
        agg = ag_r[0] + ag_r[1]
        mm = jnp.concatenate([h_r[...], agg], axis=1)
        u = _silu(_dot(mm, nw1_r[...]) + nb1_r[...])
        hn = _dot(u, nw2_r[...]) + nb2_r[...]
        if last:
            hn_r, = rest
            hn_r[...] = hn
            return
        (cw1_r, cb1_r, cw2_r, cb2_r, hn_r, cm_r) = rest
        hn_r[...] = hn
        c1 = _silu(_dot(agg, cw1_r[...]) + cb1_r[...])
        cm_r[...] = _dot(c1, cw2_r[...]) + cb2_r[...]

    grid = (n // mblk,)
    bs_m = pl.BlockSpec((mblk, h), lambda i: (i, 0))
    bs_ag = pl.BlockSpec((NC, mblk, h), lambda i: (0, i, 0))
    bs_w1 = pl.BlockSpec((2 * h, h), lambda i: (0, 0))
    bs_w = pl.BlockSpec((h, h), lambda i: (0, 0))
    bs_b = pl.BlockSpec((1, h), lambda i: (0, 0))
    bs_wc = pl.BlockSpec((h, 1), lambda i: (0, 0))
    bs_s = pl.BlockSpec((1, 1), lambda i: (0, 0))
    bs_c = pl.BlockSpec((mblk, 1), lambda i: (i, 0))
    if last:
        in_specs = [bs_m, bs_ag, bs_w1, bs_b, bs_w, bs_b]
        out_specs = bs_m
        out_shape = jax.ShapeDtypeStruct((n, h), _F32)
    else:
        in_specs = [bs_m, bs_ag, bs_w1, bs_b, bs_w, bs_b,
                    bs_w, bs_b, bs_wc, bs_s]
        out_specs = [bs_m, bs_c]
        out_shape = [jax.ShapeDtypeStruct((n, h), _F32),
                     jax.ShapeDtypeStruct((n, 1), _F32)]
    return pl.pallas_call(body, grid=grid, in_specs=in_specs,
                          out_specs=out_specs, out_shape=out_shape)


def _tc_pos_update(n):
    rows = n // LANES

    def body(px_r, py_r, pz_r, ppx_r, ppy_r, ppz_r, ox_r, oy_r, oz_r):
        ox_r[...] = px_r[...] + jnp.sum(ppx_r[...], axis=0)
        oy_r[...] = py_r[...] + jnp.sum(ppy_r[...], axis=0)
        oz_r[...] = pz_r[...] + jnp.sum(ppz_r[...], axis=0)

    bs_p = pl.BlockSpec((rows, LANES), lambda: (0, 0))
    bs_pp = pl.BlockSpec((NW, rows, LANES), lambda: (0, 0, 0))
    return pl.pallas_call(
        body,
        in_specs=[bs_p, bs_p, bs_p, bs_pp, bs_pp, bs_pp],
        out_specs=[bs_p] * 3,
        out_shape=[jax.ShapeDtypeStruct((rows, LANES), _F32)] * 3,
    )


def _tc_readout(n, h, mblk):
    nb = n // mblk

    def body(h_r, w1_r, b1_r, w2_r, b2_r, o_r, acc_r):
        @pl.when(pl.program_id(0) == 0)
        def _():
            acc_r[...] = jnp.zeros_like(acc_r)

        acc_r[...] += jnp.sum(h_r[...], axis=0, keepdims=True)

        @pl.when(pl.program_id(0) == nb - 1)
        def _():
            g = acc_r[...] * (1.0 / n)
            z = jnp.maximum(_dot(g, w1_r[...]) + b1_r[...], 0.0)
            o_r[...] = _dot(z, w2_r[...]) + b2_r[...]

    return pl.pallas_call(
        body,
        grid=(nb,),
        in_specs=[pl.BlockSpec((mblk, h), lambda i: (i, 0)),
                  pl.BlockSpec((h, h), lambda i: (0, 0)),
                  pl.BlockSpec((1, h), lambda i: (0, 0)),
                  pl.BlockSpec((h, 1), lambda i: (0, 0)),
                  pl.BlockSpec((1, 1), lambda i: (0, 0))],
        out_specs=pl.BlockSpec((1, 1), lambda i: (0, 0)),
        out_shape=jax.ShapeDtypeStruct((1, 1), _F32),
        scratch_shapes=[pltpu.VMEM((1, h), _F32)],
        compiler_params=pltpu.CompilerParams(
            dimension_semantics=("arbitrary",)),
    )


def kernel(x, pos, lattice, edge_index, enc_W, enc_b, msg_W1, msg_b1, msg_W2,
           msg_b2, coord_W1, coord_b1, coord_W2, coord_b2, node_W1, node_b1,
           node_W2, node_b2, ro_W1, ro_b1, ro_W2, ro_b2):
    n, d = x.shape
    e = edge_index.shape[1]
    h = enc_W.shape[0]
    nlayers = msg_W1.shape[0]
    mblk = 400
    beblk = 512
    assert e % (NW * ECH) == 0 and n % mblk == 0 and n % 2000 == 0
    assert e % beblk == 0 and h % LANES == 0 and n % LANES == 0

    src = edge_index[0]
    dst = edge_index[1]
    dst4s = dst.reshape(NW, e // (NW * CHI), CHI)
    src3 = src.reshape(e // ECH, SUB, CHI)
    dst3 = dst.reshape(e // ECH, SUB, CHI)
    px = pos[:, 0]
    py = pos[:, 1]
    pz = pos[:, 2]
    prow = n // LANES

    row_b = lambda v: v.reshape(1, -1)

    gather_k = _sc_gather_edges(n, e, h)
    geom_k = _sc_geometry(n, e)
    scatter_k = _sc_scatter_agg(n, e, h)
    posdel_k = _sc_pos_delta(n, e)
    enc_k = _tc_encode(n, d, h, mblk)
    edge_k = _tc_edge(e, h, beblk)
    node_k = _tc_node(n, h, mblk, last=False)
    node_last_k = _tc_node(n, h, mblk, last=True)
    posupd_k = _tc_pos_update(n)
    ro_k = _tc_readout(n, h, mblk)

    hcur = enc_k(x, enc_W.T, row_b(enc_b))
    for l in range(nlayers):
        w1p = jnp.pad(msg_W1[l].T, ((0, 3 * h - (2 * h + 1)), (0, 0)))
        gd_e, gs_e = gather_k(hcur, hcur, dst3, src3)
        drow, qx, qy, qz = geom_k(px, py, pz, dst, src)
        t2 = edge_k(gd_e, gs_e, drow, w1p, row_b(msg_b1[l]),
                    msg_W2[l].T, row_b(msg_b2[l]))
        aggp = scatter_k(t2, dst4s)
        if l < nlayers - 1:
            hcur, cm = node_k(
                hcur, aggp, node_W1[l].T, row_b(node_b1[l]),
                node_W2[l].T, row_b(node_b2[l]),
                coord_W1[l].T, row_b(coord_b1[l]),
                coord_W2[l].T, coord_b2[l].reshape(1, 1))
            ppx, ppy, ppz = posdel_k(qx, qy, qz, cm.reshape(n), src)
            pxn, pyn, pzn = posupd_k(
                px.reshape(prow, LANES), py.reshape(prow, LANES),
                pz.reshape(prow, LANES),
                ppx.reshape(NW, prow, LANES), ppy.reshape(NW, prow, LANES),
                ppz.reshape(NW, prow, LANES))
            px, py, pz = pxn.reshape(n), pyn.reshape(n), pzn.reshape(n)
        else:
            hcur = node_last_k(hcur, aggp, node_W1[l].T, row_b(node_b1[l]),
                               node_W2[l].T, row_b(node_b2[l]))
    out = ro_k(hcur, ro_W1.T, row_b(ro_b1), ro_W2.T, ro_b2.reshape(1, 1))
    return out.reshape(1)
